# baseline (speedup 1.0000x reference)
import functools

import jax
import jax.numpy as jnp
from jax import lax
from jax.experimental import pallas as pl
from jax.experimental.pallas import tpu as pltpu
from jax.experimental.pallas import tpu_sc as plsc

NC, NS, LANES = 2, 16, 16
NW = NC * NS
CHUNK = 128
IB = 16


def _sc_mesh():
    return plsc.VectorSubcoreMesh(
        core_axis_name="c", subcore_axis_name="s", num_cores=NC, num_subcores=NS
    )


N0_FRAC = 1.0


def _core_partition(total_chunks):
    n_tot = total_chunks // NS
    n0 = int(n_tot * N0_FRAC) // IB * IB
    n1 = n_tot - n0
    assert n0 % IB == 0 and n1 % IB == 0 and n0 > 0 and n1 >= 0
    return n0, n1


def _make_deg_kernel(npad, total_chunks, d):
    rows_per = npad // NS
    npw = total_chunks // NW
    assert npw % IB == 0

    @functools.partial(
        pl.kernel,
        out_type=jax.ShapeDtypeStruct((NC, npad, d), jnp.float32),
        mesh=_sc_mesh(),
        scratch_types=[
            pltpu.VMEM_SHARED((npad, d), jnp.float32),
            pltpu.VMEM((IB, CHUNK), jnp.int32),
            pltpu.VMEM((CHUNK, d), jnp.float32),
        ],
    )
    def deg_kernel(dst_hbm, zeros_hbm, ones_hbm, out_hbm, acc_sh, dstv, onesv):
        c = lax.axis_index("c")
        s = lax.axis_index("s")
        base = (c * NS + s) * npw
        r0 = s * rows_per
        pltpu.sync_copy(zeros_hbm, acc_sh.at[pl.ds(r0, rows_per)])
        pltpu.sync_copy(ones_hbm, onesv)
        plsc.subcore_barrier()

        def block(k):
            pltpu.sync_copy(dst_hbm.at[pl.ds(base + k * IB, IB)], dstv)
            for b in range(IB):
                pltpu.sync_copy(onesv, acc_sh.at[dstv.at[b]], add=True)

        pl.loop(0, npw // IB)(block)
        plsc.subcore_barrier()
        pltpu.sync_copy(
            acc_sh.at[pl.ds(r0, rows_per)], out_hbm.at[c, pl.ds(r0, rows_per)]
        )

    return deg_kernel


def _make_agg_kernel(npad, total_chunks, d):
    rows_per = npad // NS
    n0, n1 = _core_partition(total_chunks)

    @functools.partial(
        pl.kernel,
        out_type=jax.ShapeDtypeStruct((NC, npad, d), jnp.float32),
        mesh=_sc_mesh(),
        scratch_types=[
            pltpu.VMEM_SHARED((npad, d), jnp.float32),
            pltpu.VMEM((IB, CHUNK), jnp.int32),
            pltpu.VMEM((IB, CHUNK), jnp.int32),
            pltpu.VMEM((2, CHUNK, d), jnp.float32),
            pltpu.SemaphoreType.DMA,
            pltpu.SemaphoreType.DMA,
            pltpu.SemaphoreType.DMA,
            pltpu.SemaphoreType.DMA,
        ],
    )
    def agg_kernel(
        hs_hbm, src_hbm, dst_hbm, zeros_hbm, out_hbm,
        acc_sh, srcv, dstv, rows, semg0, semg1, semsc0, semsc1,
    ):
        c = lax.axis_index("c")
        s = lax.axis_index("s")
        base = jnp.where(c == 0, s * n0, NS * n0 + s * n1)
        nblocks = jnp.where(c == 0, n0 // IB, n1 // IB)
        r0 = s * rows_per
        semg = [semg0, semg1]
        semsc = [semsc0, semsc1]
        pltpu.sync_copy(zeros_hbm, acc_sh.at[pl.ds(r0, rows_per)])
        plsc.subcore_barrier()

        def drain_scatter(b):
            pltpu.make_async_copy(
                rows.at[b % 2], acc_sh.at[dstv.at[b]], semsc[b % 2]
            ).wait()

        def block(k):
            pltpu.sync_copy(src_hbm.at[pl.ds(base + k * IB, IB)], srcv)
            pltpu.sync_copy(dst_hbm.at[pl.ds(base + k * IB, IB)], dstv)
            pltpu.async_copy(hs_hbm.at[srcv.at[0]], rows.at[0], semg[0])
            for b in range(IB):
                x = b % 2
                y = (b + 1) % 2
                if b + 1 < IB:
                    if b >= 1:
                        drain_scatter(b - 1)
                    pltpu.async_copy(hs_hbm.at[srcv.at[b + 1]], rows.at[y], semg[y])
                pltpu.make_async_copy(hs_hbm.at[srcv.at[b]], rows.at[x], semg[x]).wait()
                pltpu.async_copy(rows.at[x], acc_sh.at[dstv.at[b]], semsc[x], add=True)
            drain_scatter(IB - 2)
            drain_scatter(IB - 1)

        pl.loop(0, nblocks)(block)
        plsc.subcore_barrier()
        pltpu.sync_copy(
            acc_sh.at[pl.ds(r0, rows_per)], out_hbm.at[c, pl.ds(r0, rows_per)]
        )

    return agg_kernel


def _tc_stage1(npad, bn, d_in, d_hid):
    def body(x_ref, w_ref, deg_ref, hs_ref, dv_ref):
        deg = 1.0 + deg_ref[0][:, 0:1] + deg_ref[1][:, 0:1]
        dinv = lax.rsqrt(deg)
        h = jnp.dot(x_ref[...], w_ref[...], preferred_element_type=jnp.float32)
        hs_ref[...] = dinv * h
        dv_ref[...] = jnp.broadcast_to(dinv, dv_ref.shape)

    return pl.pallas_call(
        body,
        grid=(npad // bn,),
        in_specs=[
            pl.BlockSpec((bn, d_in), lambda i: (i, 0)),
            pl.BlockSpec((d_in, d_hid), lambda i: (0, 0)),
            pl.BlockSpec((NC, bn, d_hid), lambda i: (0, i, 0)),
        ],
        out_specs=[
            pl.BlockSpec((bn, d_hid), lambda i: (i, 0)),
            pl.BlockSpec((bn, d_hid), lambda i: (i, 0)),
        ],
        out_shape=[
            jax.ShapeDtypeStruct((npad, d_hid), jnp.float32),
            jax.ShapeDtypeStruct((npad, d_hid), jnp.float32),
        ],
    )


def _tc_stage2(npad, bn, d_hid):
    def body(agg_ref, hs_ref, dv_ref, b1_ref, w2_ref, out_ref):
        agg = agg_ref[0] + agg_ref[1] + hs_ref[...]
        o1 = jnp.maximum(dv_ref[...] * agg + b1_ref[...], 0.0)
        out_ref[...] = dv_ref[...] * jnp.dot(
            o1, w2_ref[...], preferred_element_type=jnp.float32
        )

    return pl.pallas_call(
        body,
        grid=(npad // bn,),
        in_specs=[
            pl.BlockSpec((NC, bn, d_hid), lambda i: (0, i, 0)),
            pl.BlockSpec((bn, d_hid), lambda i: (i, 0)),
            pl.BlockSpec((bn, d_hid), lambda i: (i, 0)),
            pl.BlockSpec((1, d_hid), lambda i: (0, 0)),
            pl.BlockSpec((d_hid, d_hid), lambda i: (0, 0)),
        ],
        out_specs=pl.BlockSpec((bn, d_hid), lambda i: (i, 0)),
        out_shape=jax.ShapeDtypeStruct((npad, d_hid), jnp.float32),
    )


def _tc_stage3(npad, bn, d_hid, d_out):
    def body(agg_ref, hs_ref, dv_ref, b2_ref, w3_ref, b3_ref, w4_ref, b4_ref, out_ref):
        agg = agg_ref[0] + agg_ref[1] + hs_ref[...]
        o2 = jnp.maximum(dv_ref[...] * agg + b2_ref[...], 0.0)
        h3 = jnp.maximum(
            jnp.dot(o2, w3_ref[...], preferred_element_type=jnp.float32) + b3_ref[...],
            0.0,
        )
        out_ref[...] = (
            jnp.dot(h3, w4_ref[...], preferred_element_type=jnp.float32) + b4_ref[...]
        )

    return pl.pallas_call(
        body,
        grid=(npad // bn,),
        in_specs=[
            pl.BlockSpec((NC, bn, d_hid), lambda i: (0, i, 0)),
            pl.BlockSpec((bn, d_hid), lambda i: (i, 0)),
            pl.BlockSpec((bn, d_hid), lambda i: (i, 0)),
            pl.BlockSpec((1, d_hid), lambda i: (0, 0)),
            pl.BlockSpec((d_hid, d_hid), lambda i: (0, 0)),
            pl.BlockSpec((1, d_hid), lambda i: (0, 0)),
            pl.BlockSpec((d_hid, d_out), lambda i: (0, 0)),
            pl.BlockSpec((1, d_out), lambda i: (0, 0)),
        ],
        out_specs=pl.BlockSpec((bn, d_out), lambda i: (i, 0)),
        out_shape=jax.ShapeDtypeStruct((npad, d_out), jnp.float32),
    )


def kernel(x, edge_index, W1, b1, W2, b2, W3, b3, W4, b4):
    N, d_in = x.shape
    d_hid = W1.shape[1]
    d_out = W4.shape[1]
    E = edge_index.shape[1]

    bn = 1280
    npad = -(-N // bn) * bn
    epw = -(-E // (NW * IB * CHUNK)) * (IB * CHUNK)
    epad = epw * NW
    nchunks = epad // CHUNK

    src = edge_index[0].astype(jnp.int32)
    dst = edge_index[1].astype(jnp.int32)
    pad = epad - E
    src_t = jnp.concatenate([src, jnp.zeros((pad,), jnp.int32)]).reshape(
        nchunks, CHUNK
    )
    dst_t = jnp.concatenate([dst, jnp.full((pad,), N, jnp.int32)]).reshape(
        nchunks, CHUNK
    )
    x_p = jnp.pad(x, ((0, npad - N), (0, 0)))

    rows_per = npad // NS
    zerosd = jnp.zeros((rows_per, d_hid), jnp.float32)
    onesd = jnp.ones((CHUNK, d_hid), jnp.float32)

    deg = _make_deg_kernel(npad, nchunks, d_hid)(dst_t, zerosd, onesd)
    hs1, dv = _tc_stage1(npad, bn, d_in, d_hid)(x_p, W1, deg)
    agg1 = _make_agg_kernel(npad, nchunks, d_hid)(hs1, src_t, dst_t, zerosd)
    hs2 = _tc_stage2(npad, bn, d_hid)(
        agg1, hs1, dv, b1.reshape(1, d_hid), W2
    )
    agg2 = _make_agg_kernel(npad, nchunks, d_hid)(hs2, src_t, dst_t, zerosd)
    out = _tc_stage3(npad, bn, d_hid, d_out)(
        agg2,
        hs2,
        dv,
        b2.reshape(1, d_hid),
        W3,
        b3.reshape(1, d_hid),
        W4,
        b4.reshape(1, d_out),
    )
    return out[:N]


# speedup vs baseline: 1.4199x; 1.4199x over previous
import functools

import jax
import jax.numpy as jnp
from jax import lax
from jax.experimental import pallas as pl
from jax.experimental.pallas import tpu as pltpu
from jax.experimental.pallas import tpu_sc as plsc

NC, NS, LANES = 2, 16, 16
NW = NC * NS
CHUNK = 128
IB = 16


def _sc_mesh():
    return plsc.VectorSubcoreMesh(
        core_axis_name="c", subcore_axis_name="s", num_cores=NC, num_subcores=NS
    )


N0_FRAC = 0.9


def _core_partition(total_chunks):
    n_tot = total_chunks // NS
    n0 = int(n_tot * N0_FRAC) // IB * IB
    n1 = n_tot - n0
    assert n0 % IB == 0 and n1 % IB == 0 and n0 > 0 and n1 >= 0
    return n0, n1


def _make_deg_kernel(npad, total_chunks, d):
    rows_per = npad // NS
    npw = total_chunks // NW
    assert npw % IB == 0

    @functools.partial(
        pl.kernel,
        out_type=jax.ShapeDtypeStruct((NC, npad, d), jnp.float32),
        mesh=_sc_mesh(),
        scratch_types=[
            pltpu.VMEM_SHARED((npad, d), jnp.float32),
            pltpu.VMEM((IB, CHUNK), jnp.int32),
            pltpu.VMEM((CHUNK, d), jnp.float32),
        ],
    )
    def deg_kernel(dst_hbm, zeros_hbm, ones_hbm, out_hbm, acc_sh, dstv, onesv):
        c = lax.axis_index("c")
        s = lax.axis_index("s")
        base = (c * NS + s) * npw
        r0 = s * rows_per
        pltpu.sync_copy(zeros_hbm, acc_sh.at[pl.ds(r0, rows_per)])
        pltpu.sync_copy(ones_hbm, onesv)
        plsc.subcore_barrier()

        def block(k):
            pltpu.sync_copy(dst_hbm.at[pl.ds(base + k * IB, IB)], dstv)
            for b in range(IB):
                pltpu.sync_copy(onesv, acc_sh.at[dstv.at[b]], add=True)

        pl.loop(0, npw // IB)(block)
        plsc.subcore_barrier()
        pltpu.sync_copy(
            acc_sh.at[pl.ds(r0, rows_per)], out_hbm.at[c, pl.ds(r0, rows_per)]
        )

    return deg_kernel


def _make_agg_kernel(npad, total_chunks, d):
    rows_per = npad // NS
    n0, n1 = _core_partition(total_chunks)

    @functools.partial(
        pl.kernel,
        out_type=jax.ShapeDtypeStruct((NC, npad, d), jnp.float32),
        mesh=_sc_mesh(),
        scratch_types=[
            pltpu.VMEM_SHARED((npad, d), jnp.float32),
            pltpu.VMEM((IB, CHUNK), jnp.int32),
            pltpu.VMEM((IB, CHUNK), jnp.int32),
            pltpu.VMEM((2, CHUNK, d), jnp.float32),
            pltpu.SemaphoreType.DMA,
            pltpu.SemaphoreType.DMA,
            pltpu.SemaphoreType.DMA,
            pltpu.SemaphoreType.DMA,
        ],
    )
    def agg_kernel(
        hs_hbm, src_hbm, dst_hbm, zeros_hbm, out_hbm,
        acc_sh, srcv, dstv, rows, semg0, semg1, semsc0, semsc1,
    ):
        c = lax.axis_index("c")
        s = lax.axis_index("s")
        base = jnp.where(c == 0, s * n0, NS * n0 + s * n1)
        nblocks = jnp.where(c == 0, n0 // IB, n1 // IB)
        r0 = s * rows_per
        semg = [semg0, semg1]
        semsc = [semsc0, semsc1]
        pltpu.sync_copy(zeros_hbm, acc_sh.at[pl.ds(r0, rows_per)])
        plsc.subcore_barrier()

        def drain_scatter(b):
            pltpu.make_async_copy(
                rows.at[b % 2], acc_sh.at[dstv.at[b]], semsc[b % 2]
            ).wait()

        def block(k):
            pltpu.sync_copy(src_hbm.at[pl.ds(base + k * IB, IB)], srcv)
            pltpu.sync_copy(dst_hbm.at[pl.ds(base + k * IB, IB)], dstv)
            pltpu.async_copy(hs_hbm.at[srcv.at[0]], rows.at[0], semg[0])
            for b in range(IB):
                x = b % 2
                y = (b + 1) % 2
                if b + 1 < IB:
                    if b >= 1:
                        drain_scatter(b - 1)
                    pltpu.async_copy(hs_hbm.at[srcv.at[b + 1]], rows.at[y], semg[y])
                pltpu.make_async_copy(hs_hbm.at[srcv.at[b]], rows.at[x], semg[x]).wait()
                pltpu.async_copy(rows.at[x], acc_sh.at[dstv.at[b]], semsc[x], add=True)
            drain_scatter(IB - 2)
            drain_scatter(IB - 1)

        pl.loop(0, nblocks)(block)
        plsc.subcore_barrier()
        pltpu.sync_copy(
            acc_sh.at[pl.ds(r0, rows_per)], out_hbm.at[c, pl.ds(r0, rows_per)]
        )

    return agg_kernel


def _tc_stage1(npad, bn, d_in, d_hid):
    def body(x_ref, w_ref, deg_ref, hs_ref, dv_ref):
        deg = 1.0 + deg_ref[0][:, 0:1] + deg_ref[1][:, 0:1]
        dinv = lax.rsqrt(deg)
        h = jnp.dot(x_ref[...], w_ref[...], preferred_element_type=jnp.float32)
        hs_ref[...] = dinv * h
        dv_ref[...] = jnp.broadcast_to(dinv, dv_ref.shape)

    return pl.pallas_call(
        body,
        grid=(npad // bn,),
        in_specs=[
            pl.BlockSpec((bn, d_in), lambda i: (i, 0)),
            pl.BlockSpec((d_in, d_hid), lambda i: (0, 0)),
            pl.BlockSpec((NC, bn, d_hid), lambda i: (0, i, 0)),
        ],
        out_specs=[
            pl.BlockSpec((bn, d_hid), lambda i: (i, 0)),
            pl.BlockSpec((bn, d_hid), lambda i: (i, 0)),
        ],
        out_shape=[
            jax.ShapeDtypeStruct((npad, d_hid), jnp.float32),
            jax.ShapeDtypeStruct((npad, d_hid), jnp.float32),
        ],
    )


def _tc_stage2(npad, bn, d_hid):
    def body(agg_ref, hs_ref, dv_ref, b1_ref, w2_ref, out_ref):
        agg = agg_ref[0] + agg_ref[1] + hs_ref[...]
        o1 = jnp.maximum(dv_ref[...] * agg + b1_ref[...], 0.0)
        out_ref[...] = dv_ref[...] * jnp.dot(
            o1, w2_ref[...], preferred_element_type=jnp.float32
        )

    return pl.pallas_call(
        body,
        grid=(npad // bn,),
        in_specs=[
            pl.BlockSpec((NC, bn, d_hid), lambda i: (0, i, 0)),
            pl.BlockSpec((bn, d_hid), lambda i: (i, 0)),
            pl.BlockSpec((bn, d_hid), lambda i: (i, 0)),
            pl.BlockSpec((1, d_hid), lambda i: (0, 0)),
            pl.BlockSpec((d_hid, d_hid), lambda i: (0, 0)),
        ],
        out_specs=pl.BlockSpec((bn, d_hid), lambda i: (i, 0)),
        out_shape=jax.ShapeDtypeStruct((npad, d_hid), jnp.float32),
    )


def _tc_stage3(npad, bn, d_hid, d_out):
    def body(agg_ref, hs_ref, dv_ref, b2_ref, w3_ref, b3_ref, w4_ref, b4_ref, out_ref):
        agg = agg_ref[0] + agg_ref[1] + hs_ref[...]
        o2 = jnp.maximum(dv_ref[...] * agg + b2_ref[...], 0.0)
        h3 = jnp.maximum(
            jnp.dot(o2, w3_ref[...], preferred_element_type=jnp.float32) + b3_ref[...],
            0.0,
        )
        out_ref[...] = (
            jnp.dot(h3, w4_ref[...], preferred_element_type=jnp.float32) + b4_ref[...]
        )

    return pl.pallas_call(
        body,
        grid=(npad // bn,),
        in_specs=[
            pl.BlockSpec((NC, bn, d_hid), lambda i: (0, i, 0)),
            pl.BlockSpec((bn, d_hid), lambda i: (i, 0)),
            pl.BlockSpec((bn, d_hid), lambda i: (i, 0)),
            pl.BlockSpec((1, d_hid), lambda i: (0, 0)),
            pl.BlockSpec((d_hid, d_hid), lambda i: (0, 0)),
            pl.BlockSpec((1, d_hid), lambda i: (0, 0)),
            pl.BlockSpec((d_hid, d_out), lambda i: (0, 0)),
            pl.BlockSpec((1, d_out), lambda i: (0, 0)),
        ],
        out_specs=pl.BlockSpec((bn, d_out), lambda i: (i, 0)),
        out_shape=jax.ShapeDtypeStruct((npad, d_out), jnp.float32),
    )


def kernel(x, edge_index, W1, b1, W2, b2, W3, b3, W4, b4):
    N, d_in = x.shape
    d_hid = W1.shape[1]
    d_out = W4.shape[1]
    E = edge_index.shape[1]

    bn = 1280
    npad = -(-N // bn) * bn
    epw = -(-E // (NW * IB * CHUNK)) * (IB * CHUNK)
    epad = epw * NW
    nchunks = epad // CHUNK

    src = edge_index[0].astype(jnp.int32)
    dst = edge_index[1].astype(jnp.int32)
    pad = epad - E
    src_t = jnp.concatenate([src, jnp.zeros((pad,), jnp.int32)]).reshape(
        nchunks, CHUNK
    )
    dst_t = jnp.concatenate([dst, jnp.full((pad,), N, jnp.int32)]).reshape(
        nchunks, CHUNK
    )
    x_p = jnp.pad(x, ((0, npad - N), (0, 0)))

    rows_per = npad // NS
    zerosd = jnp.zeros((rows_per, d_hid), jnp.float32)
    onesd = jnp.ones((CHUNK, d_hid), jnp.float32)

    deg = _make_deg_kernel(npad, nchunks, d_hid)(dst_t, zerosd, onesd)
    hs1, dv = _tc_stage1(npad, bn, d_in, d_hid)(x_p, W1, deg)
    agg1 = _make_agg_kernel(npad, nchunks, d_hid)(hs1, src_t, dst_t, zerosd)
    hs2 = _tc_stage2(npad, bn, d_hid)(
        agg1, hs1, dv, b1.reshape(1, d_hid), W2
    )
    agg2 = _make_agg_kernel(npad, nchunks, d_hid)(hs2, src_t, dst_t, zerosd)
    out = _tc_stage3(npad, bn, d_hid, d_out)(
        agg2,
        hs2,
        dv,
        b2.reshape(1, d_hid),
        W3,
        b3.reshape(1, d_hid),
        W4,
        b4.reshape(1, d_out),
    )
    return out[:N]


# speedup vs baseline: 3.2628x; 2.2979x over previous
import functools

import jax
import jax.numpy as jnp
from jax import lax
from jax.experimental import pallas as pl
from jax.experimental.pallas import tpu as pltpu
from jax.experimental.pallas import tpu_sc as plsc

NC, NS, LANES = 2, 16, 16
NW = NC * NS
CHUNK = 128
IB = 16


def _sc_mesh():
    return plsc.VectorSubcoreMesh(
        core_axis_name="c", subcore_axis_name="s", num_cores=NC, num_subcores=NS
    )


N0_FRAC = 0.5


def _core_partition(total_chunks):
    n_tot = total_chunks // NS
    n0 = int(n_tot * N0_FRAC) // IB * IB
    n1 = n_tot - n0
    assert n0 % IB == 0 and n1 % IB == 0 and n0 > 0 and n1 >= 0
    return n0, n1


def _make_deg_kernel(npad, total_chunks, d):
    rows_per = npad // NS
    npw = total_chunks // NW
    assert npw % IB == 0

    @functools.partial(
        pl.kernel,
        out_type=jax.ShapeDtypeStruct((NC, npad, d), jnp.float32),
        mesh=_sc_mesh(),
        scratch_types=[
            pltpu.VMEM_SHARED((npad, d), jnp.float32),
            pltpu.VMEM((IB, CHUNK), jnp.int32),
            pltpu.VMEM((CHUNK, d), jnp.float32),
        ],
    )
    def deg_kernel(dst_hbm, zeros_hbm, ones_hbm, out_hbm, acc_sh, dstv, onesv):
        c = lax.axis_index("c")
        s = lax.axis_index("s")
        base = (c * NS + s) * npw
        r0 = s * rows_per
        pltpu.sync_copy(zeros_hbm, acc_sh.at[pl.ds(r0, rows_per)])
        pltpu.sync_copy(ones_hbm, onesv)
        plsc.subcore_barrier()

        def block(k):
            pltpu.sync_copy(dst_hbm.at[pl.ds(base + k * IB, IB)], dstv)
            for b in range(IB):
                pltpu.sync_copy(onesv, acc_sh.at[dstv.at[b]], add=True)

        pl.loop(0, npw // IB)(block)
        plsc.subcore_barrier()
        pltpu.sync_copy(
            acc_sh.at[pl.ds(r0, rows_per)], out_hbm.at[c, pl.ds(r0, rows_per)]
        )

    return deg_kernel


def _make_agg_kernel(npad, total_chunks, d):
    rows_per = npad // NS
    n0, n1 = _core_partition(total_chunks)

    @functools.partial(
        pl.kernel,
        out_type=jax.ShapeDtypeStruct((NC, npad, d), jnp.float32),
        mesh=_sc_mesh(),
        scratch_types=[
            pltpu.VMEM_SHARED((npad, d), jnp.float32),
            pltpu.VMEM((IB, CHUNK), jnp.int32),
            pltpu.VMEM((IB, CHUNK), jnp.int32),
            pltpu.VMEM((2, CHUNK, d), jnp.float32),
            pltpu.SemaphoreType.DMA,
            pltpu.SemaphoreType.DMA,
            pltpu.SemaphoreType.DMA,
            pltpu.SemaphoreType.DMA,
        ],
    )
    def agg_kernel(
        hs_hbm, src_hbm, dst_hbm, zeros_hbm, out_hbm,
        acc_sh, srcv, dstv, rows, semg0, semg1, semsc0, semsc1,
    ):
        c = lax.axis_index("c")
        s = lax.axis_index("s")
        base = jnp.where(c == 0, s * n0, NS * n0 + s * n1)
        nblocks = jnp.where(c == 0, n0 // IB, n1 // IB)
        r0 = s * rows_per
        semg = [semg0, semg1]
        semsc = [semsc0, semsc1]
        pltpu.sync_copy(zeros_hbm, acc_sh.at[pl.ds(r0, rows_per)])
        plsc.subcore_barrier()

        def drain_scatter(b):
            pltpu.make_async_copy(
                rows.at[b % 2], acc_sh.at[dstv.at[b]], semsc[b % 2]
            ).wait()

        def block(k):
            pltpu.sync_copy(src_hbm.at[pl.ds(base + k * IB, IB)], srcv)
            pltpu.sync_copy(dst_hbm.at[pl.ds(base + k * IB, IB)], dstv)
            pltpu.async_copy(hs_hbm.at[srcv.at[0]], rows.at[0], semg[0])
            for b in range(IB):
                x = b % 2
                y = (b + 1) % 2
                if b + 1 < IB:
                    if b >= 1:
                        drain_scatter(b - 1)
                    pltpu.async_copy(hs_hbm.at[srcv.at[b + 1]], rows.at[y], semg[y])
                pltpu.make_async_copy(hs_hbm.at[srcv.at[b]], rows.at[x], semg[x]).wait()
                pltpu.async_copy(rows.at[x], acc_sh.at[dstv.at[b]], semsc[x], add=True)
            drain_scatter(IB - 2)
            drain_scatter(IB - 1)

        pl.loop(0, nblocks)(block)
        plsc.subcore_barrier()
        pltpu.sync_copy(
            acc_sh.at[pl.ds(r0, rows_per)], out_hbm.at[c, pl.ds(r0, rows_per)]
        )

    return agg_kernel


def _tc_stage1(npad, bn, d_in, d_hid):
    def body(x_ref, w_ref, deg_ref, hs_ref, dv_ref):
        deg = 1.0 + deg_ref[0][:, 0:1] + deg_ref[1][:, 0:1]
        dinv = lax.rsqrt(deg)
        h = jnp.dot(x_ref[...], w_ref[...], preferred_element_type=jnp.float32)
        hs_ref[...] = dinv * h
        dv_ref[...] = jnp.broadcast_to(dinv, dv_ref.shape)

    return pl.pallas_call(
        body,
        grid=(npad // bn,),
        in_specs=[
            pl.BlockSpec((bn, d_in), lambda i: (i, 0)),
            pl.BlockSpec((d_in, d_hid), lambda i: (0, 0)),
            pl.BlockSpec((NC, bn, d_hid), lambda i: (0, i, 0)),
        ],
        out_specs=[
            pl.BlockSpec((bn, d_hid), lambda i: (i, 0)),
            pl.BlockSpec((bn, d_hid), lambda i: (i, 0)),
        ],
        out_shape=[
            jax.ShapeDtypeStruct((npad, d_hid), jnp.float32),
            jax.ShapeDtypeStruct((npad, d_hid), jnp.float32),
        ],
    )


def _tc_stage2(npad, bn, d_hid):
    def body(agg_ref, hs_ref, dv_ref, b1_ref, w2_ref, out_ref):
        agg = agg_ref[0] + agg_ref[1] + hs_ref[...]
        o1 = jnp.maximum(dv_ref[...] * agg + b1_ref[...], 0.0)
        out_ref[...] = dv_ref[...] * jnp.dot(
            o1, w2_ref[...], preferred_element_type=jnp.float32
        )

    return pl.pallas_call(
        body,
        grid=(npad // bn,),
        in_specs=[
            pl.BlockSpec((NC, bn, d_hid), lambda i: (0, i, 0)),
            pl.BlockSpec((bn, d_hid), lambda i: (i, 0)),
            pl.BlockSpec((bn, d_hid), lambda i: (i, 0)),
            pl.BlockSpec((1, d_hid), lambda i: (0, 0)),
            pl.BlockSpec((d_hid, d_hid), lambda i: (0, 0)),
        ],
        out_specs=pl.BlockSpec((bn, d_hid), lambda i: (i, 0)),
        out_shape=jax.ShapeDtypeStruct((npad, d_hid), jnp.float32),
    )


def _tc_stage3(npad, bn, d_hid, d_out):
    def body(agg_ref, hs_ref, dv_ref, b2_ref, w3_ref, b3_ref, w4_ref, b4_ref, out_ref):
        agg = agg_ref[0] + agg_ref[1] + hs_ref[...]
        o2 = jnp.maximum(dv_ref[...] * agg + b2_ref[...], 0.0)
        h3 = jnp.maximum(
            jnp.dot(o2, w3_ref[...], preferred_element_type=jnp.float32) + b3_ref[...],
            0.0,
        )
        out_ref[...] = (
            jnp.dot(h3, w4_ref[...], preferred_element_type=jnp.float32) + b4_ref[...]
        )

    return pl.pallas_call(
        body,
        grid=(npad // bn,),
        in_specs=[
            pl.BlockSpec((NC, bn, d_hid), lambda i: (0, i, 0)),
            pl.BlockSpec((bn, d_hid), lambda i: (i, 0)),
            pl.BlockSpec((bn, d_hid), lambda i: (i, 0)),
            pl.BlockSpec((1, d_hid), lambda i: (0, 0)),
            pl.BlockSpec((d_hid, d_hid), lambda i: (0, 0)),
            pl.BlockSpec((1, d_hid), lambda i: (0, 0)),
            pl.BlockSpec((d_hid, d_out), lambda i: (0, 0)),
            pl.BlockSpec((1, d_out), lambda i: (0, 0)),
        ],
        out_specs=pl.BlockSpec((bn, d_out), lambda i: (i, 0)),
        out_shape=jax.ShapeDtypeStruct((npad, d_out), jnp.float32),
    )


def kernel(x, edge_index, W1, b1, W2, b2, W3, b3, W4, b4):
    N, d_in = x.shape
    d_hid = W1.shape[1]
    d_out = W4.shape[1]
    E = edge_index.shape[1]

    bn = 1280
    npad = -(-N // bn) * bn
    epw = -(-E // (NW * IB * CHUNK)) * (IB * CHUNK)
    epad = epw * NW
    nchunks = epad // CHUNK

    src = edge_index[0].astype(jnp.int32)
    dst = edge_index[1].astype(jnp.int32)
    pad = epad - E
    pad_idx = jax.lax.iota(jnp.int32, pad)
    src_t = jnp.concatenate([src, pad_idx % N]).reshape(nchunks, CHUNK)
    dst_t = jnp.concatenate([dst, N + pad_idx % (npad - N)]).reshape(
        nchunks, CHUNK
    )
    x_p = jnp.pad(x, ((0, npad - N), (0, 0)))

    rows_per = npad // NS
    zerosd = jnp.zeros((rows_per, d_hid), jnp.float32)
    onesd = jnp.ones((CHUNK, d_hid), jnp.float32)

    deg = _make_deg_kernel(npad, nchunks, d_hid)(dst_t, zerosd, onesd)
    hs1, dv = _tc_stage1(npad, bn, d_in, d_hid)(x_p, W1, deg)
    agg1 = _make_agg_kernel(npad, nchunks, d_hid)(hs1, src_t, dst_t, zerosd)
    hs2 = _tc_stage2(npad, bn, d_hid)(
        agg1, hs1, dv, b1.reshape(1, d_hid), W2
    )
    agg2 = _make_agg_kernel(npad, nchunks, d_hid)(hs2, src_t, dst_t, zerosd)
    out = _tc_stage3(npad, bn, d_hid, d_out)(
        agg2,
        hs2,
        dv,
        b2.reshape(1, d_hid),
        W3,
        b3.reshape(1, d_hid),
        W4,
        b4.reshape(1, d_out),
    )
    return out[:N]
